# Initial kernel scaffold; baseline (speedup 1.0000x reference)
import jax
import jax.numpy as jnp
from jax.experimental import pallas as pl


def kernel(curr, prev, table, gate):
    raise NotImplementedError("write your pallas kernel here")


# speedup vs baseline: 1.0785x; 1.0785x over previous
import functools

import jax
import jax.numpy as jnp
import numpy as np
from jax import lax
from jax.experimental import pallas as pl
from jax.experimental.pallas import tpu as pltpu
from jax.experimental.pallas import tpu_sc as plsc

MEMORY_SIZE = 100000
NGRAM_N = 4
NUM_HEADS = 4
HEAD_DIM = 128
EMBED_DIM = NUM_HEADS * HEAD_DIM
B, W, O = 1024, 50, 50
SEQ_LEN = O + W

_WRAP_MOD = (1 << 32) % MEMORY_SIZE


def _prime_table():
    ps = []
    base = 131
    for h in range(NUM_HEADS):
        x, r = base + h * 1009, []
        for _ in range(NGRAM_N):
            r.append(x)
            x = x * 31 + 1
        ps.append(r)
    return ps


_PRIMES = _prime_table()

NC, NS = 2, 16
NW = NC * NS
QTOT = B * W
QW = QTOT // NW
B_PER = B // NW
CQ = 160
NCHUNK = QW // CQ
ROWS_CH = CQ * NUM_HEADS
NGATH = ROWS_CH // 128

@functools.lru_cache(maxsize=None)
def _build_engram_sc():
    mesh = plsc.VectorSubcoreMesh(core_axis_name="c", subcore_axis_name="s")
    return functools.partial(
        pl.kernel,
        mesh=mesh,
        out_type=jax.ShapeDtypeStruct((QTOT * NUM_HEADS, HEAD_DIM), jnp.float32),
        scratch_types=[
            pltpu.VMEM((B_PER, SEQ_LEN), jnp.int32),
            pltpu.VMEM((NUM_HEADS, HEAD_DIM), jnp.float32),
            pltpu.VMEM((NGATH, 128), jnp.int32),
            pltpu.VMEM((ROWS_CH, HEAD_DIM), jnp.float32),
            pltpu.SemaphoreType.DMA,
        ],
        compiler_params=pltpu.CompilerParams(needs_layout_passes=False),
    )(_engram_sc)


def _engram_sc(seq_hbm, table_hbm, gate_hbm, out_hbm, seq_v, g_v, idx_v, rows_v, gsem):
    wid = lax.axis_index("s") * NC + lax.axis_index("c")
    b0 = wid * B_PER

    pltpu.sync_copy(seq_hbm.at[pl.ds(b0, B_PER), :], seq_v)
    pltpu.sync_copy(gate_hbm, g_v)
    for h in range(NUM_HEADS):
        for v in range(HEAD_DIM // 16):
            sl = pl.ds(v * 16, 16)
            x = g_v[h, sl]
            g_v[h, sl] = 1.0 / (1.0 + jnp.exp(-x))

    lanes = lax.iota(jnp.int32, 16)

    def chunk_body(c, carry):
        q0 = wid * QW + c * CQ

        def hash_body(k, carry2):
            qv = q0 + k * 16 + lanes
            b = lax.div(qv, jnp.int32(W))
            w = qv - b * W
            brel = b - b0
            vals = []
            for i in range(NGRAM_N):
                col = w + (O - i)
                vals.append(plsc.load_gather(seq_v, [brel, col]))
            pos0 = ((k * 16 + lanes) * NUM_HEADS)
            for h in range(NUM_HEADS):
                hs = vals[0] * jnp.int32(_PRIMES[0][h])
                for i in range(1, NGRAM_N):
                    hs = hs + vals[i] * jnp.int32(_PRIMES[i][h])
                m = lax.rem(hs, jnp.int32(MEMORY_SIZE))
                m = jnp.where(m < 0, m + MEMORY_SIZE, m)
                m = jnp.where(hs < 0, m + _WRAP_MOD, m)
                m = jnp.where(m >= MEMORY_SIZE, m - MEMORY_SIZE, m)
                fidx = m * NUM_HEADS + h
                pos = pos0 + h
                plsc.store_scatter(idx_v, [pos >> 7, pos & 127], fidx)
            return carry2

        lax.fori_loop(0, CQ // 16, hash_body, 0)

        handles = [
            pltpu.async_copy(
                table_hbm.at[idx_v.at[j]], rows_v.at[pl.ds(j * 128, 128), :], gsem
            )
            for j in range(NGATH)
        ]
        for hd in handles:
            hd.wait()

        def mul_body(qr, carry2):
            r = qr * NUM_HEADS
            for h in range(NUM_HEADS):
                for v in range(HEAD_DIM // 16):
                    sl = pl.ds(v * 16, 16)
                    rows_v[r + h, sl] = rows_v[r + h, sl] * g_v[h, sl]
            return carry2

        lax.fori_loop(0, CQ, mul_body, 0)

        pltpu.sync_copy(rows_v, out_hbm.at[pl.ds(q0 * NUM_HEADS, ROWS_CH), :])
        return carry

    lax.fori_loop(0, NCHUNK, chunk_body, 0)


@jax.jit
def kernel(curr, prev, table, gate):
    seq = jnp.concatenate([prev, curr], axis=1)
    table_flat = table.reshape(MEMORY_SIZE * NUM_HEADS, HEAD_DIM)
    out = _build_engram_sc()(seq, table_flat, gate)
    return out.reshape(B, W, EMBED_DIM)


# speedup vs baseline: 1.9195x; 1.7799x over previous
import functools

import jax
import jax.numpy as jnp
from jax import lax
from jax.experimental import pallas as pl
from jax.experimental.pallas import tpu as pltpu
from jax.experimental.pallas import tpu_sc as plsc

MEMORY_SIZE = 100000
NGRAM_N = 4
NUM_HEADS = 4
HEAD_DIM = 128
EMBED_DIM = NUM_HEADS * HEAD_DIM
B, W, O = 1024, 50, 50
SEQ_LEN = O + W

_WRAP_MOD = (1 << 32) % MEMORY_SIZE


def _prime_table():
    ps = []
    base = 131
    for h in range(NUM_HEADS):
        x, r = base + h * 1009, []
        for _ in range(NGRAM_N):
            r.append(x)
            x = x * 31 + 1
        ps.append(r)
    return ps


_PRIMES = _prime_table()

NC, NS = 2, 16
NW = NC * NS
QTOT = B * W
QW = QTOT // NW
B_PER = B // NW
RCH = 128
NCH = QW * NUM_HEADS // RCH
NBUF = 5
PREF = 3


@functools.lru_cache(maxsize=None)
def _build_engram_sc():
    mesh = plsc.VectorSubcoreMesh(core_axis_name="c", subcore_axis_name="s")
    return functools.partial(
        pl.kernel,
        mesh=mesh,
        out_type=jax.ShapeDtypeStruct((QTOT * NUM_HEADS, HEAD_DIM), jnp.float32),
        scratch_types=[
            pltpu.VMEM((B_PER, SEQ_LEN), jnp.int32),
            pltpu.VMEM((NUM_HEADS, HEAD_DIM), jnp.float32),
            pltpu.VMEM((NCH, RCH), jnp.int32),
        ]
        + [pltpu.VMEM((RCH, HEAD_DIM), jnp.float32) for _ in range(NBUF)]
        + [pltpu.SemaphoreType.DMA for _ in range(2 * NBUF)],
        compiler_params=pltpu.CompilerParams(needs_layout_passes=False),
    )(_engram_sc)


def _engram_sc(seq_hbm, table_hbm, gate_hbm, out_hbm, seq_v, g_v, idx_v, *bufs_sems):
    bufs = bufs_sems[:NBUF]
    gsems = bufs_sems[NBUF : 2 * NBUF]
    wsems = bufs_sems[2 * NBUF :]

    wid = lax.axis_index("s") * NC + lax.axis_index("c")
    b0 = wid * B_PER
    obase = wid * QW * NUM_HEADS

    pltpu.sync_copy(seq_hbm.at[pl.ds(b0, B_PER), :], seq_v)
    pltpu.sync_copy(gate_hbm, g_v)
    for h in range(NUM_HEADS):
        for v in range(HEAD_DIM // 16):
            sl = pl.ds(v * 16, 16)
            x = g_v[h, sl]
            g_v[h, sl] = 1.0 / (1.0 + jnp.exp(-x))

    lanes = lax.iota(jnp.int32, 16)
    wq0 = wid * QW

    def hash_body(k, carry):
        qv = wq0 + k * 16 + lanes
        b = lax.div(qv, jnp.int32(W))
        w = qv - b * W
        brel = b - b0
        vals = []
        for i in range(NGRAM_N):
            col = w + (O - i)
            vals.append(plsc.load_gather(seq_v, [brel, col]))
        pos0 = (k * 16 + lanes) * NUM_HEADS
        for h in range(NUM_HEADS):
            hs = vals[0] * jnp.int32(_PRIMES[0][h])
            for i in range(1, NGRAM_N):
                hs = hs + vals[i] * jnp.int32(_PRIMES[i][h])
            m = lax.rem(hs, jnp.int32(MEMORY_SIZE))
            m = jnp.where(m < 0, m + MEMORY_SIZE, m)
            m = jnp.where(hs < 0, m + _WRAP_MOD, m)
            m = jnp.where(m >= MEMORY_SIZE, m - MEMORY_SIZE, m)
            fidx = m * NUM_HEADS + h
            pos = pos0 + h
            plsc.store_scatter(idx_v, [pos >> 7, pos & 127], fidx)
        return carry

    lax.fori_loop(0, QW // 16, hash_body, 0)

    def fire_gather(c, j):
        pltpu.async_copy(table_hbm.at[idx_v.at[c]], bufs[j], gsems[j])

    def wait_gather(j):
        pltpu.make_async_copy(table_hbm.at[pl.ds(0, RCH), :], bufs[j], gsems[j]).wait()

    def fire_write(c, j):
        pltpu.async_copy(bufs[j], out_hbm.at[pl.ds(obase + c * RCH, RCH), :], wsems[j])

    def wait_write(j):
        pltpu.make_async_copy(
            bufs[j], out_hbm.at[pl.ds(obase, RCH), :], wsems[j]
        ).wait()

    for j in range(PREF):
        fire_gather(j, j)

    gv = [
        [g_v[h, pl.ds(v * 16, 16)] for v in range(HEAD_DIM // 16)]
        for h in range(NUM_HEADS)
    ]

    def pipe_body(t, carry):
        for j in range(NBUF):
            c = t * NBUF + j
            wait_gather(j)

            buf = bufs[j]

            def mul_body(qr, carry2):
                r = qr * NUM_HEADS
                for h in range(NUM_HEADS):
                    for v in range(HEAD_DIM // 16):
                        sl = pl.ds(v * 16, 16)
                        buf[r + h, sl] = buf[r + h, sl] * gv[h][v]
                return carry2

            lax.fori_loop(0, RCH // NUM_HEADS, mul_body, 0)
            fire_write(c, j)

            jn = (j + PREF) % NBUF

            @pl.when(jnp.logical_and(c + PREF < NCH, c >= NBUF - PREF))
            def _():
                wait_write(jn)

            @pl.when(c + PREF < NCH)
            def _():
                fire_gather(c + PREF, jn)
        return carry

    lax.fori_loop(0, NCH // NBUF, pipe_body, 0)

    for j in range(NBUF):
        wait_write(j)


@jax.jit
def kernel(curr, prev, table, gate):
    seq = jnp.concatenate([prev, curr], axis=1)
    table_flat = table.reshape(MEMORY_SIZE * NUM_HEADS, HEAD_DIM)
    out = _build_engram_sc()(seq, table_flat, gate)
    return out.reshape(B, W, EMBED_DIM)
